# baseline (speedup 1.0000x reference)
import functools

import jax
import jax.numpy as jnp
from jax import lax
from jax.experimental import pallas as pl
from jax.experimental.pallas import tpu as pltpu
from jax.experimental.pallas import tpu_sc as plsc

B = 4096
E = 1000
D = 16
LANES = 16
NC = 2
NS = 16
NW = NC * NS
RPW = B // NW
CHUNK = 32
JGRP = 8
NFULL = E // LANES
TAIL = E - NFULL * LANES


def _sc_body(counts_hbm, emb_hbm, out_hbm, emb_v, cnt_v, out_v):
    wid = lax.axis_index("s") * NC + lax.axis_index("c")
    base = wid * RPW
    pltpu.sync_copy(emb_hbm, emb_v)

    def do_chunk(c, carry):
        row0 = base + c * CHUNK
        pltpu.sync_copy(counts_hbm.at[pl.ds(row0 * E, CHUNK * E)], cnt_v)

        for g in range(CHUNK // JGRP):
            def ec_step(ec, accs, g=g):
                e0 = ec * LANES
                cfs = [cnt_v[pl.ds((g * JGRP + j) * E + e0, LANES)
                             ].astype(jnp.float32) for j in range(JGRP)]
                new = list(accs)
                for l in range(LANES):
                    evec = emb_v[pl.ds((e0 + l) * D, D)]
                    for j in range(JGRP):
                        new[j] = new[j] + cfs[j][l] * evec
                return tuple(new)

            accs = tuple(jnp.zeros((D,), jnp.float32) for _ in range(JGRP))
            accs = lax.fori_loop(0, NFULL, ec_step, accs)

            e0 = E - LANES
            cfs = [cnt_v[pl.ds((g * JGRP + j) * E + e0, LANES)
                         ].astype(jnp.float32) for j in range(JGRP)]
            accs = list(accs)
            for l in range(LANES - TAIL, LANES):
                evec = emb_v[pl.ds((e0 + l) * D, D)]
                for j in range(JGRP):
                    accs[j] = accs[j] + cfs[j][l] * evec
            for j in range(JGRP):
                row = g * JGRP + j
                out_v[pl.ds((c * CHUNK + row) * D, D)] = accs[j]
        return carry

    lax.fori_loop(0, RPW // CHUNK, do_chunk, 0)
    pltpu.sync_copy(out_v, out_hbm.at[pl.ds(base * D, RPW * D)])


@functools.partial(jax.jit, static_argnames=())
def kernel(element_counts, emb):
    mesh = plsc.VectorSubcoreMesh(core_axis_name="c", subcore_axis_name="s")
    sc_fn = pl.kernel(
        _sc_body,
        out_type=jax.ShapeDtypeStruct((B * D,), jnp.float32),
        mesh=mesh,
        scratch_types=[
            pltpu.VMEM((E * D,), jnp.float32),
            pltpu.VMEM((CHUNK * E,), jnp.int32),
            pltpu.VMEM((RPW * D,), jnp.float32),
        ],
    )
    out = sc_fn(element_counts.reshape(B * E), emb.reshape(E * D))
    return out.reshape(B, D)


# speedup vs baseline: 3.4562x; 3.4562x over previous
import functools

import jax
import jax.numpy as jnp
from jax import lax
from jax.experimental import pallas as pl
from jax.experimental.pallas import tpu as pltpu
from jax.experimental.pallas import tpu_sc as plsc

B = 4096
E = 1000
D = 16
LANES = 16
NC = 2
NS = 16
NW = NC * NS

SC_ROWS = 512
TC_ROWS = B - SC_ROWS
BLK_B = 512
RPW = SC_ROWS // NW
CHUNK = RPW
JGRP = 8
NFULL = E // LANES
TAIL = E - NFULL * LANES


def _mm_kernel(counts_ref, emb_ref, out_ref):
    counts = counts_ref[:].astype(jnp.bfloat16)
    emb = emb_ref[:].astype(jnp.bfloat16)
    out_ref[:] = jnp.dot(counts, emb, preferred_element_type=jnp.float32)


def _sc_body(counts_hbm, emb_hbm, out_hbm, emb_v, cnt_v, out_v):
    wid = lax.axis_index("s") * NC + lax.axis_index("c")
    base = TC_ROWS + wid * RPW
    pltpu.sync_copy(emb_hbm, emb_v)

    def do_chunk(c, carry):
        row0 = base + c * CHUNK
        pltpu.sync_copy(counts_hbm.at[pl.ds(row0 * E, CHUNK * E)], cnt_v)

        for g in range(CHUNK // JGRP):
            def ec_step(ec, accs, g=g):
                e0 = ec * LANES
                cfs = [cnt_v[pl.ds((g * JGRP + j) * E + e0, LANES)
                             ].astype(jnp.float32) for j in range(JGRP)]
                new = list(accs)
                for l in range(LANES):
                    evec = emb_v[pl.ds((e0 + l) * D, D)]
                    for j in range(JGRP):
                        new[j] = new[j] + cfs[j][l] * evec
                return tuple(new)

            accs = tuple(jnp.zeros((D,), jnp.float32) for _ in range(JGRP))
            accs = lax.fori_loop(0, NFULL, ec_step, accs)

            e0 = E - LANES
            cfs = [cnt_v[pl.ds((g * JGRP + j) * E + e0, LANES)
                         ].astype(jnp.float32) for j in range(JGRP)]
            accs = list(accs)
            for l in range(LANES - TAIL, LANES):
                evec = emb_v[pl.ds((e0 + l) * D, D)]
                for j in range(JGRP):
                    accs[j] = accs[j] + cfs[j][l] * evec
            for j in range(JGRP):
                row = g * JGRP + j
                out_v[pl.ds((c * CHUNK + row) * D, D)] = accs[j]
        return carry

    lax.fori_loop(0, RPW // CHUNK, do_chunk, 0)
    pltpu.sync_copy(out_v, out_hbm.at[pl.ds((base - TC_ROWS) * D, RPW * D)])


@functools.partial(jax.jit, static_argnames=())
def kernel(element_counts, emb):
    mesh = plsc.VectorSubcoreMesh(core_axis_name="c", subcore_axis_name="s")
    sc_fn = pl.kernel(
        _sc_body,
        out_type=jax.ShapeDtypeStruct((SC_ROWS * D,), jnp.float32),
        mesh=mesh,
        scratch_types=[
            pltpu.VMEM((E * D,), jnp.float32),
            pltpu.VMEM((CHUNK * E,), jnp.int32),
            pltpu.VMEM((RPW * D,), jnp.float32),
        ],
    )
    out_sc = sc_fn(element_counts.reshape(B * E), emb.reshape(E * D))

    out_tc = pl.pallas_call(
        _mm_kernel,
        grid=(TC_ROWS // BLK_B,),
        in_specs=[
            pl.BlockSpec((BLK_B, E), lambda i: (i, 0)),
            pl.BlockSpec((E, D), lambda i: (0, 0)),
        ],
        out_specs=pl.BlockSpec((BLK_B, D), lambda i: (i, 0)),
        out_shape=jax.ShapeDtypeStruct((TC_ROWS, D), jnp.float32),
    )(element_counts, emb)

    return jnp.concatenate([out_tc, out_sc.reshape(SC_ROWS, D)], axis=0)


# speedup vs baseline: 4.2730x; 1.2363x over previous
import functools

import jax
import jax.numpy as jnp
from jax import lax
from jax.experimental import pallas as pl
from jax.experimental.pallas import tpu as pltpu
from jax.experimental.pallas import tpu_sc as plsc

B = 4096
E = 1000
D = 16
LANES = 16
NC = 2
NS = 16
NW = NC * NS

SC_ROWS = 256
TC_ROWS = B - SC_ROWS
BLK_B = 512
RPW = SC_ROWS // NW
CHUNK = RPW
JGRP = 8
NFULL = E // LANES
TAIL = E - NFULL * LANES


def _mm_kernel(counts_ref, emb_ref, out_ref):
    counts = counts_ref[:].astype(jnp.bfloat16)
    emb = emb_ref[:].astype(jnp.bfloat16)
    out_ref[:] = jnp.dot(counts, emb, preferred_element_type=jnp.float32)


def _sc_body(counts_hbm, emb_hbm, out_hbm, emb_v, cnt_v, out_v):
    wid = lax.axis_index("s") * NC + lax.axis_index("c")
    base = TC_ROWS + wid * RPW
    pltpu.sync_copy(emb_hbm, emb_v)

    def do_chunk(c, carry):
        row0 = base + c * CHUNK
        pltpu.sync_copy(counts_hbm.at[pl.ds(row0 * E, CHUNK * E)], cnt_v)

        for g in range(CHUNK // JGRP):
            def ec_step(ec, accs, g=g):
                e0 = ec * LANES
                cfs = [cnt_v[pl.ds((g * JGRP + j) * E + e0, LANES)
                             ].astype(jnp.float32) for j in range(JGRP)]
                new = list(accs)
                for l in range(LANES):
                    evec = emb_v[pl.ds((e0 + l) * D, D)]
                    for j in range(JGRP):
                        new[j] = new[j] + cfs[j][l] * evec
                return tuple(new)

            accs = tuple(jnp.zeros((D,), jnp.float32) for _ in range(JGRP))
            accs = lax.fori_loop(0, NFULL, ec_step, accs)

            e0 = E - LANES
            cfs = [cnt_v[pl.ds((g * JGRP + j) * E + e0, LANES)
                         ].astype(jnp.float32) for j in range(JGRP)]
            accs = list(accs)
            for l in range(LANES - TAIL, LANES):
                evec = emb_v[pl.ds((e0 + l) * D, D)]
                for j in range(JGRP):
                    accs[j] = accs[j] + cfs[j][l] * evec
            for j in range(JGRP):
                row = g * JGRP + j
                out_v[pl.ds((c * CHUNK + row) * D, D)] = accs[j]
        return carry

    lax.fori_loop(0, RPW // CHUNK, do_chunk, 0)
    pltpu.sync_copy(out_v, out_hbm.at[pl.ds((base - TC_ROWS) * D, RPW * D)])


@functools.partial(jax.jit, static_argnames=())
def kernel(element_counts, emb):
    mesh = plsc.VectorSubcoreMesh(core_axis_name="c", subcore_axis_name="s")
    sc_fn = pl.kernel(
        _sc_body,
        out_type=jax.ShapeDtypeStruct((SC_ROWS * D,), jnp.float32),
        mesh=mesh,
        scratch_types=[
            pltpu.VMEM((E * D,), jnp.float32),
            pltpu.VMEM((CHUNK * E,), jnp.int32),
            pltpu.VMEM((RPW * D,), jnp.float32),
        ],
        cost_estimate=pl.CostEstimate(
            flops=2 * SC_ROWS * E * D,
            transcendentals=0,
            bytes_accessed=SC_ROWS * E * 4 + E * D * 4 + SC_ROWS * D * 4,
        ),
    )
    out_sc = sc_fn(element_counts.reshape(B * E), emb.reshape(E * D))

    out_tc = pl.pallas_call(
        _mm_kernel,
        grid=(TC_ROWS // BLK_B,),
        in_specs=[
            pl.BlockSpec((BLK_B, E), lambda i: (i, 0)),
            pl.BlockSpec((E, D), lambda i: (0, 0)),
        ],
        out_specs=pl.BlockSpec((BLK_B, D), lambda i: (i, 0)),
        out_shape=jax.ShapeDtypeStruct((TC_ROWS, D), jnp.float32),
    )(element_counts, emb)

    return jnp.concatenate([out_tc, out_sc.reshape(SC_ROWS, D)], axis=0)


# speedup vs baseline: 11.7879x; 2.7587x over previous
import functools

import jax
import jax.numpy as jnp
from jax.experimental import pallas as pl


BLK_B = 4096


def _mm_kernel(counts_ref, emb_ref, out_ref):
    counts = counts_ref[:].astype(jnp.bfloat16)
    emb = emb_ref[:].astype(jnp.bfloat16)
    out_ref[:] = jnp.dot(counts, emb, preferred_element_type=jnp.float32)


@functools.partial(jax.jit, static_argnames=())
def kernel(element_counts, emb):
    B, E = element_counts.shape
    D = emb.shape[1]
    grid = (B // BLK_B,)
    return pl.pallas_call(
        _mm_kernel,
        grid=grid,
        in_specs=[
            pl.BlockSpec((BLK_B, E), lambda i: (i, 0)),
            pl.BlockSpec((E, D), lambda i: (0, 0)),
        ],
        out_specs=pl.BlockSpec((BLK_B, D), lambda i: (i, 0)),
        out_shape=jax.ShapeDtypeStruct((B, D), jnp.float32),
    )(element_counts, emb)


# speedup vs baseline: 12.3940x; 1.0514x over previous
import functools

import jax
import jax.numpy as jnp
from jax.experimental import pallas as pl


BLK_B = 2048


def _mm_kernel(counts_ref, emb_ref, out_ref):
    counts = counts_ref[:].astype(jnp.bfloat16)
    emb = emb_ref[:].astype(jnp.bfloat16)
    out_ref[:] = jnp.dot(counts, emb, preferred_element_type=jnp.float32)


@functools.partial(jax.jit, static_argnames=())
def kernel(element_counts, emb):
    B, E = element_counts.shape
    D = emb.shape[1]
    grid = (B // BLK_B,)
    return pl.pallas_call(
        _mm_kernel,
        grid=grid,
        in_specs=[
            pl.BlockSpec((BLK_B, E), lambda i: (i, 0)),
            pl.BlockSpec((E, D), lambda i: (0, 0)),
        ],
        out_specs=pl.BlockSpec((BLK_B, D), lambda i: (i, 0)),
        out_shape=jax.ShapeDtypeStruct((B, D), jnp.float32),
    )(element_counts, emb)
